# baseline (speedup 1.0000x reference)
import functools

import jax
import jax.numpy as jnp
from jax import lax
from jax.experimental import pallas as pl
from jax.experimental.pallas import tpu as pltpu
from jax.experimental.pallas import tpu_sc as plsc

_NC = 2
_NS = 16
_NW = _NC * _NS
_CH = 32


def kernel(x, pos_table):
    b, s, h = x.shape
    s_per_w = s // _NW
    n_chunks = s_per_w // _CH
    lanes = 16
    groups = h // lanes

    mesh = plsc.VectorSubcoreMesh(core_axis_name="c", subcore_axis_name="s")

    @functools.partial(
        pl.kernel,
        mesh=mesh,
        out_type=jax.ShapeDtypeStruct((b, s, h), jnp.float32),
        scratch_types=[
            pltpu.VMEM((_CH, h), jnp.float32),
            pltpu.VMEM((_CH, h), jnp.float32),
        ],
    )
    def sc_add(x_hbm, pos_hbm, out_hbm, pbuf, xbuf):
        wid = lax.axis_index("s") * _NC + lax.axis_index("c")
        s_base = wid * s_per_w

        def chunk_body(i, carry):
            s0 = s_base + i * _CH
            pltpu.sync_copy(pos_hbm.at[pl.ds(s0, _CH)], pbuf)
            for bb in range(b):
                pltpu.sync_copy(x_hbm.at[bb, pl.ds(s0, _CH)], xbuf)

                def row_body(r, c2):
                    for g in range(groups):
                        plsc.addupdate(
                            xbuf.at[r, pl.ds(g * lanes, lanes)],
                            pbuf[r, pl.ds(g * lanes, lanes)],
                        )
                    return c2

                lax.fori_loop(0, _CH, row_body, 0)
                pltpu.sync_copy(xbuf, out_hbm.at[bb, pl.ds(s0, _CH)])
            return carry

        lax.fori_loop(0, n_chunks, chunk_body, 0)

    return sc_add(x, pos_table)


# speedup vs baseline: 1.2636x; 1.2636x over previous
import functools

import jax
import jax.numpy as jnp
from jax import lax
from jax.experimental import pallas as pl
from jax.experimental.pallas import tpu as pltpu
from jax.experimental.pallas import tpu_sc as plsc

_NC = 2
_NS = 16
_NW = _NC * _NS
_CH = 16
_LANES = 16


def kernel(x, pos_table):
    b, s, h = x.shape
    s_per_w = s // _NW
    n_chunks = s_per_w // _CH
    n_steps = n_chunks * b
    groups = h // _LANES

    mesh = plsc.VectorSubcoreMesh(core_axis_name="c", subcore_axis_name="s")

    @functools.partial(
        pl.kernel,
        mesh=mesh,
        out_type=jax.ShapeDtypeStruct((b, s, h), jnp.float32),
        scratch_types=[
            pltpu.VMEM((2, _CH, h), jnp.float32),
            pltpu.VMEM((2, _CH, h), jnp.float32),
            pltpu.SemaphoreType.DMA,
            pltpu.SemaphoreType.DMA,
            pltpu.SemaphoreType.DMA,
            pltpu.SemaphoreType.DMA,
            pltpu.SemaphoreType.DMA,
            pltpu.SemaphoreType.DMA,
        ],
    )
    def sc_add(x_hbm, pos_hbm, out_hbm, xbuf, pbuf,
               xsem0, xsem1, osem0, osem1, psem0, psem1):
        xsems = (xsem0, xsem1)
        osems = (osem0, osem1)
        psems = (psem0, psem1)
        wid = lax.axis_index("s") * _NC + lax.axis_index("c")
        s_base = wid * s_per_w

        def x_copy(t, k):
            c = t // b
            bb = t - c * b
            s0 = s_base + c * _CH
            return pltpu.make_async_copy(
                x_hbm.at[bb, pl.ds(s0, _CH)], xbuf.at[k], xsems[k])

        def out_copy(t, k):
            c = t // b
            bb = t - c * b
            s0 = s_base + c * _CH
            return pltpu.make_async_copy(
                xbuf.at[k], out_hbm.at[bb, pl.ds(s0, _CH)], osems[k])

        def pos_copy(c, k):
            s0 = s_base + c * _CH
            return pltpu.make_async_copy(
                pos_hbm.at[pl.ds(s0, _CH)], pbuf.at[k], psems[k])

        x_copy(0, 0).start()
        pos_copy(0, 0).start()

        def pair_body(it, carry):
            t_base = it * 2 * b
            for j in range(2 * b):
                k = j % 2
                pb = j // b
                t = t_base + j
                c = t // b
                if j % b == 0:
                    @pl.when(c + 1 < n_chunks)
                    def _():
                        pos_copy(c + 1, 1 - pb).start()
                    pos_copy(c, pb).wait()
                x_copy(t, k).wait()
                @pl.when(t >= 1)
                def _():
                    out_copy(t - 1, 1 - k).wait()

                @pl.when(t + 1 < n_steps)
                def _():
                    x_copy(t + 1, 1 - k).start()

                def row_body(r, c2):
                    for g in range(groups):
                        plsc.addupdate(
                            xbuf.at[k, r, pl.ds(g * _LANES, _LANES)],
                            pbuf[pb, r, pl.ds(g * _LANES, _LANES)],
                        )
                    return c2

                lax.fori_loop(0, _CH, row_body, 0)
                out_copy(t, k).start()
            return carry

        lax.fori_loop(0, n_steps // (2 * b), pair_body, 0)
        out_copy(n_steps - 1, (n_steps - 1) % 2).wait()

    return sc_add(x, pos_table)


# speedup vs baseline: 2.6358x; 2.0860x over previous
import functools

import jax
import jax.numpy as jnp
from jax import lax
from jax.experimental import pallas as pl
from jax.experimental.pallas import tpu as pltpu
from jax.experimental.pallas import tpu_sc as plsc

_NC = 2
_NS = 16
_NW = _NC * _NS
_CH = 16
_LANES = 16


def kernel(x, pos_table):
    b, s, h = x.shape
    s_per_w = s // _NW
    n_chunks = s_per_w // _CH
    n_steps = n_chunks * b
    groups = h // _LANES

    mesh = plsc.VectorSubcoreMesh(core_axis_name="c", subcore_axis_name="s")

    @functools.partial(
        pl.kernel,
        mesh=mesh,
        out_type=jax.ShapeDtypeStruct((b, s, h), jnp.float32),
        scratch_types=[
            pltpu.VMEM((2, _CH, h), jnp.float32),
            pltpu.VMEM((2, _CH, h), jnp.float32),
            pltpu.SemaphoreType.DMA,
            pltpu.SemaphoreType.DMA,
            pltpu.SemaphoreType.DMA,
            pltpu.SemaphoreType.DMA,
            pltpu.SemaphoreType.DMA,
            pltpu.SemaphoreType.DMA,
        ],
    )
    def sc_add(x_hbm, pos_hbm, out_hbm, xbuf, pbuf,
               xsem0, xsem1, osem0, osem1, psem0, psem1):
        xsems = (xsem0, xsem1)
        osems = (osem0, osem1)
        psems = (psem0, psem1)
        wid = lax.axis_index("s") * _NC + lax.axis_index("c")
        s_base = wid * s_per_w

        def x_copy(t, k):
            c = t // b
            bb = t - c * b
            s0 = s_base + c * _CH
            return pltpu.make_async_copy(
                x_hbm.at[bb, pl.ds(s0, _CH)], xbuf.at[k], xsems[k])

        def out_copy(t, k):
            c = t // b
            bb = t - c * b
            s0 = s_base + c * _CH
            return pltpu.make_async_copy(
                xbuf.at[k], out_hbm.at[bb, pl.ds(s0, _CH)], osems[k])

        def pos_copy(c, k):
            s0 = s_base + c * _CH
            return pltpu.make_async_copy(
                pos_hbm.at[pl.ds(s0, _CH)], pbuf.at[k], psems[k])

        x_copy(0, 0).start()
        pos_copy(0, 0).start()

        def pair_body(it, carry):
            t_base = it * 2 * b
            for j in range(2 * b):
                k = j % 2
                pb = j // b
                t = t_base + j
                c = t // b
                if j % b == 0:
                    @pl.when(c + 1 < n_chunks)
                    def _():
                        pos_copy(c + 1, 1 - pb).start()
                    pos_copy(c, pb).wait()
                x_copy(t, k).wait()
                @pl.when(t >= 1)
                def _():
                    out_copy(t - 1, 1 - k).wait()

                @pl.when(t + 1 < n_steps)
                def _():
                    x_copy(t + 1, 1 - k).start()

                @plsc.parallel_loop(0, _CH * h, _LANES, unroll=8)
                def _(i):
                    r = i // h
                    g = i - r * h
                    plsc.addupdate(
                        xbuf.at[k, r, pl.ds(g, _LANES)],
                        pbuf[pb, r, pl.ds(g, _LANES)],
                    )
                out_copy(t, k).start()
            return carry

        lax.fori_loop(0, n_steps // (2 * b), pair_body, 0)
        out_copy(n_steps - 1, (n_steps - 1) % 2).wait()

    return sc_add(x, pos_table)
